# Initial kernel scaffold; baseline (speedup 1.0000x reference)
import jax
import jax.numpy as jnp
from jax.experimental import pallas as pl


def kernel(edge_index, emb, W1, b1, W2, b2):
    raise NotImplementedError("write your pallas kernel here")


# speedup vs baseline: 7.0797x; 7.0797x over previous
import functools

import jax
import jax.numpy as jnp
from jax import lax
from jax.experimental import pallas as pl
from jax.experimental.pallas import tpu as pltpu
from jax.experimental.pallas import tpu_sc as plsc

N = 10000
D = 128
E = 320000

NC = 2
NS = 16
NW = NC * NS
COLS = D // NW
VCHUNK = 2000
ACHUNK = 2560

_sc_mesh = plsc.VectorSubcoreMesh(core_axis_name="c", subcore_axis_name="s")
_sc_params = pltpu.CompilerParams(needs_layout_passes=False)


@functools.partial(
    pl.kernel,
    out_type=jax.ShapeDtypeStruct((NW, N), jnp.float32),
    mesh=_sc_mesh,
    compiler_params=_sc_params,
    scratch_types=[
        pltpu.VMEM((N,), jnp.float32),
        pltpu.VMEM((VCHUNK,), jnp.int32),
    ],
)
def _deg_kernel(dst_hbm, parts_hbm, hist, idx):
    wid = lax.axis_index("s") * NC + lax.axis_index("c")
    per_tile = E // NW
    ones = jnp.ones((16,), jnp.float32)

    def zero(j, _):
        hist[pl.ds(j * 16, 16)] = jnp.zeros((16,), jnp.float32)
        return _

    lax.fori_loop(0, N // 16, zero, None)

    def chunk(k, _):
        pltpu.sync_copy(dst_hbm.at[pl.ds(wid * per_tile + k * VCHUNK, VCHUNK)], idx)

        def grp(g, __):
            d = idx[pl.ds(g * 16, 16)]
            plsc.addupdate_scatter(hist, [d], ones)
            return __

        lax.fori_loop(0, VCHUNK // 16, grp, None)
        return _

    lax.fori_loop(0, per_tile // VCHUNK, chunk, None)
    pltpu.sync_copy(hist, parts_hbm.at[wid])


def _dinv_body(parts_ref, dinv_ref):
    s = jnp.sum(parts_ref[...], axis=0, keepdims=True) + 1.0
    dinv_ref[...] = lax.rsqrt(s)


def _dinv(parts):
    return pl.pallas_call(
        _dinv_body,
        out_shape=jax.ShapeDtypeStruct((1, N), jnp.float32),
    )(parts)


@functools.partial(
    pl.kernel,
    out_type=jax.ShapeDtypeStruct((D, N), jnp.float32),
    mesh=_sc_mesh,
    compiler_params=_sc_params,
    scratch_types=[
        pltpu.VMEM((COLS, N), jnp.float32),
        pltpu.VMEM((COLS, N), jnp.float32),
        pltpu.VMEM((N,), jnp.float32),
        pltpu.VMEM((ACHUNK,), jnp.int32),
        pltpu.VMEM((ACHUNK,), jnp.int32),
        pltpu.VMEM((COLS, 16), jnp.float32),
    ],
)
def _agg_kernel(yt_hbm, dinv_hbm, src_hbm, dst_hbm, bias_hbm, zt_hbm,
                y, acc, dinv, sbuf, dbuf, bias):
    wid = lax.axis_index("s") * NC + lax.axis_index("c")
    row0 = wid * COLS
    pltpu.sync_copy(dinv_hbm.at[0], dinv)
    pltpu.sync_copy(yt_hbm.at[pl.ds(row0, COLS)], y)
    pltpu.sync_copy(bias_hbm.at[pl.ds(row0, COLS)], bias)

    def pro(j, _):
        dv = dinv[pl.ds(j * 16, 16)]
        for c in range(COLS):
            y[c, pl.ds(j * 16, 16)] = y[c, pl.ds(j * 16, 16)] * dv
            acc[c, pl.ds(j * 16, 16)] = jnp.zeros((16,), jnp.float32)
        return _

    lax.fori_loop(0, N // 16, pro, None)

    colidx = [jnp.full((16,), c, jnp.int32) for c in range(COLS)]

    def chunk(k, _):
        pltpu.sync_copy(src_hbm.at[pl.ds(k * ACHUNK, ACHUNK)], sbuf)
        pltpu.sync_copy(dst_hbm.at[pl.ds(k * ACHUNK, ACHUNK)], dbuf)

        def grp(g, __):
            s = sbuf[pl.ds(g * 16, 16)]
            d = dbuf[pl.ds(g * 16, 16)]
            for c in range(COLS):
                v = plsc.load_gather(y, [colidx[c], s])
                plsc.addupdate_scatter(acc, [colidx[c], d], v)
            return __

        lax.fori_loop(0, ACHUNK // 16, grp, None)
        return _

    lax.fori_loop(0, E // ACHUNK, chunk, None)

    def epi(j, _):
        dv = dinv[pl.ds(j * 16, 16)]
        for c in range(COLS):
            b = bias[c, :]
            sl = pl.ds(j * 16, 16)
            acc[c, sl] = (acc[c, sl] + y[c, sl]) * dv + b
        return _

    lax.fori_loop(0, N // 16, epi, None)
    pltpu.sync_copy(acc, zt_hbm.at[pl.ds(row0, COLS)])


_MM_BLOCK = 1000


def _mm_body(z_ref, w1_ref, b1_ref, w2_ref, g_ref):
    h = jnp.dot(z_ref[...], w1_ref[...], preferred_element_type=jnp.float32)
    h = jnp.maximum(h + b1_ref[...], 0.0)
    g_ref[...] = jnp.dot(h, w2_ref[...], preferred_element_type=jnp.float32)


def _mm(z, W1, b1, W2):
    return pl.pallas_call(
        _mm_body,
        grid=(N // _MM_BLOCK,),
        in_specs=[
            pl.BlockSpec((_MM_BLOCK, D), lambda i: (i, 0)),
            pl.BlockSpec((D, 2 * D), lambda i: (0, 0)),
            pl.BlockSpec((1, 2 * D), lambda i: (0, 0)),
            pl.BlockSpec((2 * D, D), lambda i: (0, 0)),
        ],
        out_specs=pl.BlockSpec((_MM_BLOCK, D), lambda i: (i, 0)),
        out_shape=jax.ShapeDtypeStruct((N, D), jnp.float32),
    )(z, W1, b1, W2)


def kernel(edge_index, emb, W1, b1, W2, b2):
    src = edge_index[0]
    dst = edge_index[1]

    parts = _deg_kernel(dst)
    dinv = _dinv(parts)

    zeros_bias = jnp.zeros((D, 16), jnp.float32)
    b2_bcast = jnp.broadcast_to(b2.reshape(D, 1), (D, 16))
    z1t = _agg_kernel(emb.T, dinv, src, dst, zeros_bias)
    g = _mm(z1t.T, W1, b1.reshape(1, 2 * D), W2)
    z2t = _agg_kernel(g.T, dinv, src, dst, b2_bcast)
    return z2t.T


# speedup vs baseline: 13.8994x; 1.9633x over previous
import functools

import jax
import jax.numpy as jnp
from jax import lax
from jax.experimental import pallas as pl
from jax.experimental.pallas import tpu as pltpu
from jax.experimental.pallas import tpu_sc as plsc

N = 10000
D = 128
E = 320000

NC = 2
NS = 16
NW = NC * NS
COLS = D // NW
VCHUNK = 2000
ACHUNK = 6400

_sc_mesh = plsc.VectorSubcoreMesh(core_axis_name="c", subcore_axis_name="s")
_sc_params = pltpu.CompilerParams(needs_layout_passes=False)


@functools.partial(
    pl.kernel,
    out_type=jax.ShapeDtypeStruct((NW, N), jnp.float32),
    mesh=_sc_mesh,
    compiler_params=_sc_params,
    scratch_types=[
        pltpu.VMEM((N,), jnp.float32),
        pltpu.VMEM((VCHUNK,), jnp.int32),
    ],
)
def _deg_kernel(dst_hbm, parts_hbm, hist, idx):
    wid = lax.axis_index("s") * NC + lax.axis_index("c")
    per_tile = E // NW
    ones = jnp.ones((16,), jnp.float32)

    def zero(j, _):
        hist[pl.ds(j * 16, 16)] = jnp.zeros((16,), jnp.float32)
        return _

    lax.fori_loop(0, N // 16, zero, None)

    def chunk(k, _):
        pltpu.sync_copy(dst_hbm.at[pl.ds(wid * per_tile + k * VCHUNK, VCHUNK)], idx)

        def grp(g, __):
            d = idx[pl.ds(g * 16, 16)]
            plsc.addupdate_scatter(hist, [d], ones)
            return __

        lax.fori_loop(0, VCHUNK // 16, grp, None)
        return _

    lax.fori_loop(0, per_tile // VCHUNK, chunk, None)
    pltpu.sync_copy(hist, parts_hbm.at[wid])


def _dinv_body(parts_ref, dinv_ref):
    s = jnp.sum(parts_ref[...], axis=0, keepdims=True) + 1.0
    dinv_ref[...] = lax.rsqrt(s)


def _dinv(parts):
    return pl.pallas_call(
        _dinv_body,
        out_shape=jax.ShapeDtypeStruct((1, N), jnp.float32),
    )(parts)


@functools.partial(
    pl.kernel,
    out_type=jax.ShapeDtypeStruct((D, N), jnp.float32),
    mesh=_sc_mesh,
    compiler_params=_sc_params,
    scratch_types=[
        pltpu.VMEM((COLS, N), jnp.float32),
        pltpu.VMEM((COLS, N), jnp.float32),
        pltpu.VMEM((N,), jnp.float32),
        pltpu.VMEM((2, 2, ACHUNK), jnp.int32),
        pltpu.SemaphoreType.DMA((2,)),
        pltpu.VMEM((COLS, 16), jnp.float32),
    ],
)
def _agg_kernel(yt_hbm, dinv_hbm, edge_hbm, bias_hbm, zt_hbm,
                y, acc, dinv, ebuf, sem, bias):
    wid = lax.axis_index("s") * NC + lax.axis_index("c")
    row0 = wid * COLS
    pltpu.sync_copy(dinv_hbm.at[0], dinv)
    pltpu.sync_copy(yt_hbm.at[pl.ds(row0, COLS)], y)
    pltpu.sync_copy(bias_hbm.at[pl.ds(row0, COLS)], bias)

    def pro(j, _):
        dv = dinv[pl.ds(j * 16, 16)]
        for c in range(COLS):
            y[c, pl.ds(j * 16, 16)] = y[c, pl.ds(j * 16, 16)] * dv
            acc[c, pl.ds(j * 16, 16)] = jnp.zeros((16,), jnp.float32)
        return _

    lax.fori_loop(0, N // 16, pro, None)

    colidx = [jnp.full((16,), c, jnp.int32) for c in range(COLS)]
    nch = E // ACHUNK

    pltpu.async_copy(edge_hbm.at[:, pl.ds(0, ACHUNK)], ebuf.at[0], sem.at[0])

    def chunk(k, _):
        slot = lax.rem(k, 2)
        nxt = lax.rem(k + 1, 2)

        @pl.when(k + 1 < nch)
        def _prefetch():
            pltpu.async_copy(edge_hbm.at[:, pl.ds((k + 1) * ACHUNK, ACHUNK)],
                             ebuf.at[nxt], sem.at[nxt])

        pltpu.make_async_copy(edge_hbm.at[:, pl.ds(k * ACHUNK, ACHUNK)],
                              ebuf.at[slot], sem.at[slot]).wait()

        def grp(g, __):
            s = ebuf[slot, 0, pl.ds(g * 16, 16)]
            d = ebuf[slot, 1, pl.ds(g * 16, 16)]
            vals = [plsc.load_gather(y, [colidx[c], s]) for c in range(COLS)]
            for c in range(COLS):
                plsc.addupdate_scatter(acc, [colidx[c], d], vals[c])
            return __

        lax.fori_loop(0, ACHUNK // 16, grp, None, unroll=2)
        return _

    lax.fori_loop(0, nch, chunk, None)

    def epi(j, _):
        dv = dinv[pl.ds(j * 16, 16)]
        for c in range(COLS):
            b = bias[c, :]
            sl = pl.ds(j * 16, 16)
            acc[c, sl] = (acc[c, sl] + y[c, sl]) * dv + b
        return _

    lax.fori_loop(0, N // 16, epi, None)
    pltpu.sync_copy(acc, zt_hbm.at[pl.ds(row0, COLS)])


_MM_BLOCK = 1000


def _mm_body(z_ref, w1_ref, b1_ref, w2_ref, g_ref):
    h = jnp.dot(z_ref[...], w1_ref[...], preferred_element_type=jnp.float32)
    h = jnp.maximum(h + b1_ref[...], 0.0)
    g_ref[...] = jnp.dot(h, w2_ref[...], preferred_element_type=jnp.float32)


def _mm(z, W1, b1, W2):
    return pl.pallas_call(
        _mm_body,
        grid=(N // _MM_BLOCK,),
        in_specs=[
            pl.BlockSpec((_MM_BLOCK, D), lambda i: (i, 0)),
            pl.BlockSpec((D, 2 * D), lambda i: (0, 0)),
            pl.BlockSpec((1, 2 * D), lambda i: (0, 0)),
            pl.BlockSpec((2 * D, D), lambda i: (0, 0)),
        ],
        out_specs=pl.BlockSpec((_MM_BLOCK, D), lambda i: (i, 0)),
        out_shape=jax.ShapeDtypeStruct((N, D), jnp.float32),
    )(z, W1, b1, W2)


def kernel(edge_index, emb, W1, b1, W2, b2):
    src = edge_index[0]
    dst = edge_index[1]

    parts = _deg_kernel(dst)
    dinv = _dinv(parts)

    zeros_bias = jnp.zeros((D, 16), jnp.float32)
    b2_bcast = jnp.broadcast_to(b2.reshape(D, 1), (D, 16))
    z1t = _agg_kernel(emb.T, dinv, edge_index, zeros_bias)
    g = _mm(z1t.T, W1, b1.reshape(1, 2 * D), W2)
    z2t = _agg_kernel(g.T, dinv, edge_index, b2_bcast)
    return z2t.T


# speedup vs baseline: 21.1860x; 1.5242x over previous
import functools

import jax
import jax.numpy as jnp
from jax import lax
from jax.experimental import pallas as pl
from jax.experimental.pallas import tpu as pltpu
from jax.experimental.pallas import tpu_sc as plsc

N = 10000
D = 128
E = 320000

NC = 2
NS = 16
NW = NC * NS
COLS = D // NW
VCHUNK = 2000
ACHUNK = 6400

_sc_mesh = plsc.VectorSubcoreMesh(core_axis_name="c", subcore_axis_name="s")
_sc_params = pltpu.CompilerParams(needs_layout_passes=False)


@functools.partial(
    pl.kernel,
    out_type=jax.ShapeDtypeStruct((NW, N), jnp.float32),
    mesh=_sc_mesh,
    compiler_params=_sc_params,
    scratch_types=[
        pltpu.VMEM((N,), jnp.float32),
        pltpu.VMEM((VCHUNK,), jnp.int32),
    ],
)
def _deg_kernel(dst_hbm, parts_hbm, hist, idx):
    wid = lax.axis_index("s") * NC + lax.axis_index("c")
    per_tile = E // NW
    ones = jnp.ones((16,), jnp.float32)

    def zero(j, _):
        hist[pl.ds(j * 16, 16)] = jnp.zeros((16,), jnp.float32)
        return _

    lax.fori_loop(0, N // 16, zero, None)

    def chunk(k, _):
        pltpu.sync_copy(dst_hbm.at[pl.ds(wid * per_tile + k * VCHUNK, VCHUNK)], idx)

        def grp(g, __):
            d = idx[pl.ds(g * 16, 16)]
            plsc.addupdate_scatter(hist, [d], ones)
            return __

        lax.fori_loop(0, VCHUNK // 16, grp, None)
        return _

    lax.fori_loop(0, per_tile // VCHUNK, chunk, None)
    pltpu.sync_copy(hist, parts_hbm.at[wid])


def _dinv_body(parts_ref, dinv_ref):
    s = jnp.sum(parts_ref[...], axis=0, keepdims=True) + 1.0
    dinv_ref[...] = lax.rsqrt(s)


def _dinv(parts):
    return pl.pallas_call(
        _dinv_body,
        out_shape=jax.ShapeDtypeStruct((1, N), jnp.float32),
    )(parts)


@functools.partial(
    pl.kernel,
    out_type=jax.ShapeDtypeStruct((D, N), jnp.float32),
    mesh=_sc_mesh,
    compiler_params=_sc_params,
    scratch_types=[
        pltpu.VMEM((COLS, N), jnp.float32),
        pltpu.VMEM((COLS, N), jnp.float32),
        pltpu.VMEM((N,), jnp.float32),
        pltpu.VMEM((2, 2, ACHUNK), jnp.int32),
        pltpu.SemaphoreType.DMA((2,)),
        pltpu.VMEM((COLS, 16), jnp.float32),
    ],
)
def _agg_kernel(yt_hbm, dinv_hbm, edge_hbm, bias_hbm, zt_hbm,
                y, acc, dinv, ebuf, sem, bias):
    wid = lax.axis_index("s") * NC + lax.axis_index("c")
    row0 = wid * COLS
    pltpu.sync_copy(dinv_hbm.at[0], dinv)
    pltpu.sync_copy(yt_hbm.at[pl.ds(row0, COLS)], y)
    pltpu.sync_copy(bias_hbm.at[pl.ds(row0, COLS)], bias)

    def pro(j, _):
        dv = dinv[pl.ds(j * 16, 16)]
        for c in range(COLS):
            y[c, pl.ds(j * 16, 16)] = y[c, pl.ds(j * 16, 16)] * dv
            acc[c, pl.ds(j * 16, 16)] = jnp.zeros((16,), jnp.float32)
        return _

    lax.fori_loop(0, N // 16, pro, None)

    colidx = [jnp.full((16,), c, jnp.int32) for c in range(COLS)]
    nch = E // ACHUNK

    pltpu.async_copy(edge_hbm.at[:, pl.ds(0, ACHUNK)], ebuf.at[0], sem.at[0])

    def chunk(k, _):
        slot = lax.rem(k, 2)
        nxt = lax.rem(k + 1, 2)

        @pl.when(k + 1 < nch)
        def _prefetch():
            pltpu.async_copy(edge_hbm.at[:, pl.ds((k + 1) * ACHUNK, ACHUNK)],
                             ebuf.at[nxt], sem.at[nxt])

        pltpu.make_async_copy(edge_hbm.at[:, pl.ds(k * ACHUNK, ACHUNK)],
                              ebuf.at[slot], sem.at[slot]).wait()

        @plsc.parallel_loop(0, ACHUNK // 16, unroll=4)
        def grp(g):
            s = ebuf[slot, 0, pl.ds(g * 16, 16)]
            d = ebuf[slot, 1, pl.ds(g * 16, 16)]
            vals = [plsc.load_gather(y, [colidx[c], s]) for c in range(COLS)]
            for c in range(COLS):
                plsc.addupdate_scatter(acc, [colidx[c], d], vals[c])

        return _

    lax.fori_loop(0, nch, chunk, None)

    def epi(j, _):
        dv = dinv[pl.ds(j * 16, 16)]
        for c in range(COLS):
            b = bias[c, :]
            sl = pl.ds(j * 16, 16)
            acc[c, sl] = (acc[c, sl] + y[c, sl]) * dv + b
        return _

    lax.fori_loop(0, N // 16, epi, None)
    pltpu.sync_copy(acc, zt_hbm.at[pl.ds(row0, COLS)])


_MM_BLOCK = 1000


def _mm_body(z_ref, w1_ref, b1_ref, w2_ref, g_ref):
    h = jnp.dot(z_ref[...], w1_ref[...], preferred_element_type=jnp.float32)
    h = jnp.maximum(h + b1_ref[...], 0.0)
    g_ref[...] = jnp.dot(h, w2_ref[...], preferred_element_type=jnp.float32)


def _mm(z, W1, b1, W2):
    return pl.pallas_call(
        _mm_body,
        grid=(N // _MM_BLOCK,),
        in_specs=[
            pl.BlockSpec((_MM_BLOCK, D), lambda i: (i, 0)),
            pl.BlockSpec((D, 2 * D), lambda i: (0, 0)),
            pl.BlockSpec((1, 2 * D), lambda i: (0, 0)),
            pl.BlockSpec((2 * D, D), lambda i: (0, 0)),
        ],
        out_specs=pl.BlockSpec((_MM_BLOCK, D), lambda i: (i, 0)),
        out_shape=jax.ShapeDtypeStruct((N, D), jnp.float32),
    )(z, W1, b1, W2)


def kernel(edge_index, emb, W1, b1, W2, b2):
    src = edge_index[0]
    dst = edge_index[1]

    parts = _deg_kernel(dst)
    dinv = _dinv(parts)

    zeros_bias = jnp.zeros((D, 16), jnp.float32)
    b2_bcast = jnp.broadcast_to(b2.reshape(D, 1), (D, 16))
    z1t = _agg_kernel(emb.T, dinv, edge_index, zeros_bias)
    g = _mm(z1t.T, W1, b1.reshape(1, 2 * D), W2)
    z2t = _agg_kernel(g.T, dinv, edge_index, b2_bcast)
    return z2t.T


# speedup vs baseline: 23.0448x; 1.0877x over previous
import functools

import jax
import jax.numpy as jnp
from jax import lax
from jax.experimental import pallas as pl
from jax.experimental.pallas import tpu as pltpu
from jax.experimental.pallas import tpu_sc as plsc

N = 10000
D = 128
E = 320000

NC = 2
NS = 16
NW = NC * NS
COLS = D // NW
VCHUNK = 1280
ACHUNK = 6400

_sc_mesh = plsc.VectorSubcoreMesh(core_axis_name="c", subcore_axis_name="s")
_sc_params = pltpu.CompilerParams(needs_layout_passes=False)


@functools.partial(
    pl.kernel,
    out_type=(
        jax.ShapeDtypeStruct((NW, N), jnp.float32),
        jax.ShapeDtypeStruct((E,), jnp.int32),
    ),
    mesh=_sc_mesh,
    compiler_params=_sc_params,
    scratch_types=[
        pltpu.VMEM((N,), jnp.float32),
        pltpu.VMEM((2, VCHUNK), jnp.int32),
        pltpu.VMEM((VCHUNK,), jnp.int32),
    ],
)
def _deg_kernel(edge_hbm, parts_hbm, packed_hbm, hist, ebuf, pbuf):
    wid = lax.axis_index("s") * NC + lax.axis_index("c")
    nchunks = E // VCHUNK
    ones = jnp.ones((16,), jnp.float32)

    @plsc.parallel_loop(0, N // 16, unroll=2)
    def zero(j):
        hist[pl.ds(j * 16, 16)] = jnp.zeros((16,), jnp.float32)

    def chunk(j, _):
        cid = wid + j * NW

        @pl.when(cid < nchunks)
        def _do():
            pltpu.sync_copy(edge_hbm.at[:, pl.ds(cid * VCHUNK, VCHUNK)], ebuf)

            @plsc.parallel_loop(0, VCHUNK // 16, unroll=4)
            def grp(g):
                s = ebuf[0, pl.ds(g * 16, 16)]
                d = ebuf[1, pl.ds(g * 16, 16)]
                plsc.addupdate_scatter(hist, [d], ones)
                pbuf[pl.ds(g * 16, 16)] = (d << 16) | s

            pltpu.sync_copy(pbuf, packed_hbm.at[pl.ds(cid * VCHUNK, VCHUNK)])

        return _

    lax.fori_loop(0, (nchunks + NW - 1) // NW, chunk, None)
    pltpu.sync_copy(hist, parts_hbm.at[wid])


def _dinv_body(parts_ref, dinv_ref):
    s = jnp.sum(parts_ref[...], axis=0, keepdims=True) + 1.0
    dinv_ref[...] = lax.rsqrt(s)


def _dinv(parts):
    return pl.pallas_call(
        _dinv_body,
        out_shape=jax.ShapeDtypeStruct((1, N), jnp.float32),
    )(parts)


@functools.partial(
    pl.kernel,
    out_type=jax.ShapeDtypeStruct((D, N), jnp.float32),
    mesh=_sc_mesh,
    compiler_params=_sc_params,
    scratch_types=[
        pltpu.VMEM((COLS, N), jnp.float32),
        pltpu.VMEM((COLS, N), jnp.float32),
        pltpu.VMEM((N,), jnp.float32),
        pltpu.VMEM((2, ACHUNK), jnp.int32),
        pltpu.SemaphoreType.DMA((2,)),
        pltpu.VMEM((COLS, 16), jnp.float32),
    ],
)
def _agg_kernel(yt_hbm, dinv_hbm, packed_hbm, bias_hbm, zt_hbm,
                y, acc, dinv, ebuf, sem, bias):
    wid = lax.axis_index("s") * NC + lax.axis_index("c")
    row0 = wid * COLS
    pltpu.sync_copy(dinv_hbm.at[0], dinv)
    pltpu.sync_copy(yt_hbm.at[pl.ds(row0, COLS)], y)
    pltpu.sync_copy(bias_hbm.at[pl.ds(row0, COLS)], bias)

    @plsc.parallel_loop(0, N // 16, unroll=2)
    def pro(j):
        dv = dinv[pl.ds(j * 16, 16)]
        for c in range(COLS):
            y[c, pl.ds(j * 16, 16)] = y[c, pl.ds(j * 16, 16)] * dv
            acc[c, pl.ds(j * 16, 16)] = jnp.zeros((16,), jnp.float32)

    colidx = [jnp.full((16,), c, jnp.int32) for c in range(COLS)]
    nch = E // ACHUNK

    pltpu.async_copy(packed_hbm.at[pl.ds(0, ACHUNK)], ebuf.at[0], sem.at[0])

    def chunk(k, _):
        slot = lax.rem(k, 2)
        nxt = lax.rem(k + 1, 2)

        @pl.when(k + 1 < nch)
        def _prefetch():
            pltpu.async_copy(packed_hbm.at[pl.ds((k + 1) * ACHUNK, ACHUNK)],
                             ebuf.at[nxt], sem.at[nxt])

        pltpu.make_async_copy(packed_hbm.at[pl.ds(k * ACHUNK, ACHUNK)],
                              ebuf.at[slot], sem.at[slot]).wait()

        @plsc.parallel_loop(0, ACHUNK // 16, unroll=4)
        def grp(g):
            w = ebuf[slot, pl.ds(g * 16, 16)]
            s = w & 0xFFFF
            d = lax.shift_right_logical(w, 16)
            vals = [plsc.load_gather(y, [colidx[c], s]) for c in range(COLS)]
            for c in range(COLS):
                plsc.addupdate_scatter(acc, [colidx[c], d], vals[c])

        return _

    lax.fori_loop(0, nch, chunk, None)

    @plsc.parallel_loop(0, N // 16, unroll=2)
    def epi(j):
        dv = dinv[pl.ds(j * 16, 16)]
        for c in range(COLS):
            b = bias[c, :]
            sl = pl.ds(j * 16, 16)
            acc[c, sl] = (acc[c, sl] + y[c, sl]) * dv + b
    pltpu.sync_copy(acc, zt_hbm.at[pl.ds(row0, COLS)])


_MM_BLOCK = 1000


def _mm_body(z_ref, w1_ref, b1_ref, w2_ref, g_ref):
    h = jnp.dot(z_ref[...], w1_ref[...], preferred_element_type=jnp.float32)
    h = jnp.maximum(h + b1_ref[...], 0.0)
    g_ref[...] = jnp.dot(h, w2_ref[...], preferred_element_type=jnp.float32)


def _mm(z, W1, b1, W2):
    return pl.pallas_call(
        _mm_body,
        grid=(N // _MM_BLOCK,),
        in_specs=[
            pl.BlockSpec((_MM_BLOCK, D), lambda i: (i, 0)),
            pl.BlockSpec((D, 2 * D), lambda i: (0, 0)),
            pl.BlockSpec((1, 2 * D), lambda i: (0, 0)),
            pl.BlockSpec((2 * D, D), lambda i: (0, 0)),
        ],
        out_specs=pl.BlockSpec((_MM_BLOCK, D), lambda i: (i, 0)),
        out_shape=jax.ShapeDtypeStruct((N, D), jnp.float32),
    )(z, W1, b1, W2)


def kernel(edge_index, emb, W1, b1, W2, b2):
    parts, packed = _deg_kernel(edge_index)
    dinv = _dinv(parts)

    zeros_bias = jnp.zeros((D, 16), jnp.float32)
    b2_bcast = jnp.broadcast_to(b2.reshape(D, 1), (D, 16))
    z1t = _agg_kernel(emb.T, dinv, packed, zeros_bias)
    g = _mm(z1t.T, W1, b1.reshape(1, 2 * D), W2)
    z2t = _agg_kernel(g.T, dinv, packed, b2_bcast)
    return z2t.T


# speedup vs baseline: 23.8299x; 1.0341x over previous
import functools

import jax
import jax.numpy as jnp
from jax import lax
from jax.experimental import pallas as pl
from jax.experimental.pallas import tpu as pltpu
from jax.experimental.pallas import tpu_sc as plsc

N = 10000
D = 128
E = 320000

NC = 2
NS = 16
NW = NC * NS
COLS = D // NW
VCHUNK = 1280
ACHUNK = 6400

_sc_mesh = plsc.VectorSubcoreMesh(core_axis_name="c", subcore_axis_name="s")
_sc_params = pltpu.CompilerParams(needs_layout_passes=False)


@functools.partial(
    pl.kernel,
    out_type=(
        jax.ShapeDtypeStruct((NW, N), jnp.float32),
        jax.ShapeDtypeStruct((E,), jnp.int32),
    ),
    mesh=_sc_mesh,
    compiler_params=_sc_params,
    scratch_types=[
        pltpu.VMEM((N,), jnp.float32),
        pltpu.VMEM((2, VCHUNK), jnp.int32),
        pltpu.VMEM((VCHUNK,), jnp.int32),
    ],
)
def _deg_kernel(edge_hbm, parts_hbm, packed_hbm, hist, ebuf, pbuf):
    wid = lax.axis_index("s") * NC + lax.axis_index("c")
    nchunks = E // VCHUNK
    ones = jnp.ones((16,), jnp.float32)

    @plsc.parallel_loop(0, N // 16, unroll=2)
    def zero(j):
        hist[pl.ds(j * 16, 16)] = jnp.zeros((16,), jnp.float32)

    def chunk(j, _):
        cid = wid + j * NW

        @pl.when(cid < nchunks)
        def _do():
            pltpu.sync_copy(edge_hbm.at[:, pl.ds(cid * VCHUNK, VCHUNK)], ebuf)

            @plsc.parallel_loop(0, VCHUNK // 16, unroll=4)
            def grp(g):
                s = ebuf[0, pl.ds(g * 16, 16)]
                d = ebuf[1, pl.ds(g * 16, 16)]
                plsc.addupdate_scatter(hist, [d], ones)
                pbuf[pl.ds(g * 16, 16)] = (d << 16) | s

            pltpu.sync_copy(pbuf, packed_hbm.at[pl.ds(cid * VCHUNK, VCHUNK)])

        return _

    lax.fori_loop(0, (nchunks + NW - 1) // NW, chunk, None)
    pltpu.sync_copy(hist, parts_hbm.at[wid])


def _dinv_body(parts_ref, dinv_ref):
    s = jnp.sum(parts_ref[...], axis=0, keepdims=True) + 1.0
    dinv_ref[...] = lax.rsqrt(s)


def _dinv(parts):
    return pl.pallas_call(
        _dinv_body,
        out_shape=jax.ShapeDtypeStruct((1, N), jnp.float32),
    )(parts)


@functools.partial(
    pl.kernel,
    out_type=jax.ShapeDtypeStruct((D, N), jnp.float32),
    mesh=_sc_mesh,
    compiler_params=_sc_params,
    scratch_types=[
        pltpu.VMEM((COLS, N), jnp.float32),
        pltpu.VMEM((COLS, N), jnp.float32),
        pltpu.VMEM((N,), jnp.float32),
        pltpu.VMEM((2, ACHUNK), jnp.int32),
        pltpu.SemaphoreType.DMA((2,)),
        pltpu.VMEM((COLS, 16), jnp.float32),
    ],
)
def _agg_kernel(yt_hbm, dinv_hbm, packed_hbm, bias_hbm, zt_hbm,
                y, acc, dinv, ebuf, sem, bias):
    wid = lax.axis_index("s") * NC + lax.axis_index("c")
    row0 = wid * COLS
    pltpu.sync_copy(dinv_hbm.at[0], dinv)
    pltpu.sync_copy(yt_hbm.at[pl.ds(row0, COLS)], y)
    pltpu.sync_copy(bias_hbm.at[pl.ds(row0, COLS)], bias)

    @plsc.parallel_loop(0, N // 16, unroll=2)
    def pro(j):
        dv = dinv[pl.ds(j * 16, 16)]
        for c in range(COLS):
            y[c, pl.ds(j * 16, 16)] = y[c, pl.ds(j * 16, 16)] * dv
            acc[c, pl.ds(j * 16, 16)] = jnp.zeros((16,), jnp.float32)

    colidx = [jnp.full((16,), c, jnp.int32) for c in range(COLS)]
    nch = E // ACHUNK

    pltpu.async_copy(packed_hbm.at[pl.ds(0, ACHUNK)], ebuf.at[0], sem.at[0])

    def chunk(k, _):
        slot = lax.rem(k, 2)
        nxt = lax.rem(k + 1, 2)

        @pl.when(k + 1 < nch)
        def _prefetch():
            pltpu.async_copy(packed_hbm.at[pl.ds((k + 1) * ACHUNK, ACHUNK)],
                             ebuf.at[nxt], sem.at[nxt])

        pltpu.make_async_copy(packed_hbm.at[pl.ds(k * ACHUNK, ACHUNK)],
                              ebuf.at[slot], sem.at[slot]).wait()

        @plsc.parallel_loop(0, ACHUNK // 16, unroll=4)
        def grp(g):
            w = ebuf[slot, pl.ds(g * 16, 16)]
            s = w & 0xFFFF
            d = lax.shift_right_logical(w, 16)
            vals = [plsc.load_gather(y, [colidx[c], s]) for c in range(COLS)]
            for c in range(COLS):
                plsc.addupdate_scatter(acc, [colidx[c], d], vals[c])

        return _

    lax.fori_loop(0, nch, chunk, None)

    @plsc.parallel_loop(0, N // 16, unroll=2)
    def epi(j):
        dv = dinv[pl.ds(j * 16, 16)]
        for c in range(COLS):
            b = bias[c, :]
            sl = pl.ds(j * 16, 16)
            acc[c, sl] = (acc[c, sl] + y[c, sl]) * dv + b
    pltpu.sync_copy(acc, zt_hbm.at[pl.ds(row0, COLS)])


_MMT_BLOCK = 2000


def _mm_t_body(zt_ref, w1_ref, b1_ref, w2_ref, gt_ref):
    cdims = (((0,), (0,)), ((), ()))
    h = lax.dot_general(w1_ref[...], zt_ref[...], cdims,
                        preferred_element_type=jnp.float32)
    h = jnp.maximum(h + b1_ref[...], 0.0)
    gt_ref[...] = lax.dot_general(w2_ref[...], h, cdims,
                                  preferred_element_type=jnp.float32)


def _mm_t(zt, W1, b1col, W2):
    return pl.pallas_call(
        _mm_t_body,
        out_shape=jax.ShapeDtypeStruct((D, N), jnp.float32),
    )(zt, W1, b1col, W2)


def kernel(edge_index, emb, W1, b1, W2, b2):
    parts, packed = _deg_kernel(edge_index)
    dinv = _dinv(parts)

    zeros_bias = jnp.zeros((D, 16), jnp.float32)
    b2_bcast = jnp.broadcast_to(b2.reshape(D, 1), (D, 16))
    z1t = _agg_kernel(emb.T, dinv, packed, zeros_bias)
    gt = _mm_t(z1t, W1, b1.reshape(2 * D, 1), W2)
    z2t = _agg_kernel(gt, dinv, packed, b2_bcast)
    return z2t.T


# speedup vs baseline: 27.8169x; 1.1673x over previous
import functools

import jax
import jax.numpy as jnp
from jax import lax
from jax.experimental import pallas as pl
from jax.experimental.pallas import tpu as pltpu
from jax.experimental.pallas import tpu_sc as plsc

N = 10000
D = 128
E = 320000

NC = 2
NS = 16
NW = NC * NS
COLS = D // NW
VCHUNK = 1280
ACHUNK = 6400

_sc_mesh = plsc.VectorSubcoreMesh(core_axis_name="c", subcore_axis_name="s")
_sc_params = pltpu.CompilerParams(needs_layout_passes=False)


@functools.partial(
    pl.kernel,
    out_type=(
        jax.ShapeDtypeStruct((NW, N), jnp.float32),
        jax.ShapeDtypeStruct((E,), jnp.int32),
    ),
    mesh=_sc_mesh,
    compiler_params=_sc_params,
    scratch_types=[
        pltpu.VMEM((N,), jnp.float32),
        pltpu.VMEM((2, VCHUNK), jnp.int32),
        pltpu.VMEM((VCHUNK,), jnp.int32),
    ],
)
def _deg_kernel(edge_hbm, parts_hbm, packed_hbm, hist, ebuf, pbuf):
    wid = lax.axis_index("s") * NC + lax.axis_index("c")
    nchunks = E // VCHUNK
    ones = jnp.ones((16,), jnp.float32)

    @plsc.parallel_loop(0, N // 16, unroll=2)
    def zero(j):
        hist[pl.ds(j * 16, 16)] = jnp.zeros((16,), jnp.float32)

    def chunk(j, _):
        cid = wid + j * NW

        @pl.when(cid < nchunks)
        def _do():
            pltpu.sync_copy(edge_hbm.at[:, pl.ds(cid * VCHUNK, VCHUNK)], ebuf)

            @plsc.parallel_loop(0, VCHUNK // 16, unroll=4)
            def grp(g):
                s = ebuf[0, pl.ds(g * 16, 16)]
                d = ebuf[1, pl.ds(g * 16, 16)]
                plsc.addupdate_scatter(hist, [d], ones)
                pbuf[pl.ds(g * 16, 16)] = (d << 16) | s

            pltpu.sync_copy(pbuf, packed_hbm.at[pl.ds(cid * VCHUNK, VCHUNK)])

        return _

    lax.fori_loop(0, (nchunks + NW - 1) // NW, chunk, None)
    pltpu.sync_copy(hist, parts_hbm.at[wid])


def _dinv_body(parts_ref, dinv_ref):
    s = jnp.sum(parts_ref[...], axis=0, keepdims=True) + 1.0
    dinv_ref[...] = lax.rsqrt(s)


def _dinv(parts):
    return pl.pallas_call(
        _dinv_body,
        out_shape=jax.ShapeDtypeStruct((1, N), jnp.float32),
    )(parts)


CPAIRS = COLS // 2
CSTAGE = 2000


@functools.partial(
    pl.kernel,
    out_type=jax.ShapeDtypeStruct((D, N), jnp.float32),
    mesh=_sc_mesh,
    compiler_params=_sc_params,
    scratch_types=[
        pltpu.VMEM((CPAIRS, N), jnp.int32),
        pltpu.VMEM((COLS, N), jnp.float32),
        pltpu.VMEM((N,), jnp.float32),
        pltpu.VMEM((2, ACHUNK), jnp.int32),
        pltpu.SemaphoreType.DMA((2,)),
        pltpu.VMEM((COLS, 16), jnp.float32),
        pltpu.VMEM((COLS, N), jnp.float32),
    ],
)
def _agg_kernel(yt_hbm, dinv_hbm, packed_hbm, bias_hbm, zt_hbm,
                yp, acc, dinv, ebuf, sem, bias, stage):
    wid = lax.axis_index("s") * NC + lax.axis_index("c")
    row0 = wid * COLS
    pltpu.sync_copy(dinv_hbm.at[0], dinv)
    pltpu.sync_copy(bias_hbm.at[pl.ds(row0, COLS)], bias)

    pltpu.sync_copy(yt_hbm.at[pl.ds(row0, COLS)], stage)

    @plsc.parallel_loop(0, N // 16, unroll=2)
    def pro(j):
        sl = pl.ds(j * 16, 16)
        dv = dinv[sl]
        for p in range(CPAIRS):
            a = stage[2 * p, sl] * dv
            bb = stage[2 * p + 1, sl] * dv
            pk = plsc.pack(a, bb, format=plsc.PackFormat.INTERLEAVED)
            yp[p, sl] = plsc.bitcast(pk, jnp.int32)
        for c in range(COLS):
            acc[c, sl] = jnp.zeros((16,), jnp.float32)

    colidx = [jnp.full((16,), c, jnp.int32) for c in range(COLS)]
    pairidx = [jnp.full((16,), p, jnp.int32) for p in range(CPAIRS)]
    nch = E // ACHUNK

    pltpu.async_copy(packed_hbm.at[pl.ds(0, ACHUNK)], ebuf.at[0], sem.at[0])

    def chunk(k, _):
        slot = lax.rem(k, 2)
        nxt = lax.rem(k + 1, 2)

        @pl.when(k + 1 < nch)
        def _prefetch():
            pltpu.async_copy(packed_hbm.at[pl.ds((k + 1) * ACHUNK, ACHUNK)],
                             ebuf.at[nxt], sem.at[nxt])

        pltpu.make_async_copy(packed_hbm.at[pl.ds(k * ACHUNK, ACHUNK)],
                              ebuf.at[slot], sem.at[slot]).wait()

        @plsc.parallel_loop(0, ACHUNK // 16, unroll=4)
        def grp(g):
            w = ebuf[slot, pl.ds(g * 16, 16)]
            s = w & 0xFFFF
            d = lax.shift_right_logical(w, 16)
            vals = []
            for p in range(CPAIRS):
                pw = plsc.load_gather(yp, [pairidx[p], s])
                pb = plsc.bitcast(pw, jnp.bfloat16)
                a, bb = plsc.unpack(pb, format=plsc.PackFormat.INTERLEAVED)
                vals += [a, bb]
            for c in range(COLS):
                plsc.addupdate_scatter(acc, [colidx[c], d], vals[c])

        return _

    lax.fori_loop(0, nch, chunk, None)

    @plsc.parallel_loop(0, N // 16, unroll=2)
    def epi(j):
        sl = pl.ds(j * 16, 16)
        dv = dinv[sl]
        for p in range(CPAIRS):
            pb = plsc.bitcast(yp[p, sl], jnp.bfloat16)
            a, bb = plsc.unpack(pb, format=plsc.PackFormat.INTERLEAVED)
            acc[2 * p, sl] = (acc[2 * p, sl] + a) * dv + bias[2 * p, :]
            acc[2 * p + 1, sl] = ((acc[2 * p + 1, sl] + bb) * dv
                                  + bias[2 * p + 1, :])
    pltpu.sync_copy(acc, zt_hbm.at[pl.ds(row0, COLS)])


_MMT_BLOCK = 2000


def _mm_t_body(zt_ref, w1_ref, b1_ref, w2_ref, gt_ref):
    cdims = (((0,), (0,)), ((), ()))
    h = lax.dot_general(w1_ref[...], zt_ref[...], cdims,
                        preferred_element_type=jnp.float32)
    h = jnp.maximum(h + b1_ref[...], 0.0)
    gt_ref[...] = lax.dot_general(w2_ref[...], h, cdims,
                                  preferred_element_type=jnp.float32)


def _mm_t(zt, W1, b1col, W2):
    return pl.pallas_call(
        _mm_t_body,
        out_shape=jax.ShapeDtypeStruct((D, N), jnp.float32),
    )(zt, W1, b1col, W2)


def kernel(edge_index, emb, W1, b1, W2, b2):
    parts, packed = _deg_kernel(edge_index)
    dinv = _dinv(parts)

    zeros_bias = jnp.zeros((D, 16), jnp.float32)
    b2_bcast = jnp.broadcast_to(b2.reshape(D, 1), (D, 16))
    z1t = _agg_kernel(emb.T, dinv, packed, zeros_bias)
    gt = _mm_t(z1t, W1, b1.reshape(2 * D, 1), W2)
    z2t = _agg_kernel(gt, dinv, packed, b2_bcast)
    return z2t.T
